# Initial kernel scaffold; baseline (speedup 1.0000x reference)
import jax
import jax.numpy as jnp
from jax.experimental import pallas as pl


def kernel(hidden, feature_bank, expert_bank_idx, ln_gamma, ln_beta, rW1, rb1, rW2, rb2, We1, be1, We2, be2, We3, be3, alpha):
    raise NotImplementedError("write your pallas kernel here")


# speedup vs baseline: 1.6659x; 1.6659x over previous
import functools

import jax
import jax.numpy as jnp
from jax.experimental import pallas as pl

B = 2048
D = 2048
E = 8
NC = 16
FB = 16
FPE = 2
H = 1024
RH = 1024
K = 2

LOGIT_PAD = 128

TBA = 256
TBB = 1024


def _gelu_exact(x):
    return x * 0.5 * (1.0 + jax.lax.erf(x * 0.7071067811865476))


def _router_kernel(hid_ref, feat_ref, g_ref, b_ref, rw1a_ref, rw1b_ref,
                   rb1_ref, rw2_ref, rb2_ref, h_ref, w_ref):
    x = hid_ref[...]
    mu = jnp.mean(x, axis=-1, keepdims=True)
    var = jnp.mean((x - mu) ** 2, axis=-1, keepdims=True)
    h = (x - mu) / jnp.sqrt(var + 1e-5) * g_ref[...] + b_ref[...]
    r1 = jnp.dot(h, rw1a_ref[...], preferred_element_type=jnp.float32)
    r1 = r1 + jnp.dot(feat_ref[...], rw1b_ref[...],
                      preferred_element_type=jnp.float32)
    r1 = _gelu_exact(r1 + rb1_ref[...])
    logits = jnp.dot(r1, rw2_ref[...], preferred_element_type=jnp.float32)
    logits = logits + rb2_ref[...]
    iota = jax.lax.broadcasted_iota(jnp.int32, logits.shape, 1)
    v1 = jnp.max(logits, axis=-1, keepdims=True)
    i1 = jnp.min(jnp.where(logits == v1, iota, LOGIT_PAD), axis=-1,
                 keepdims=True)
    masked = jnp.where(iota == i1, -jnp.inf, logits)
    v2 = jnp.max(masked, axis=-1, keepdims=True)
    i2 = jnp.min(jnp.where(masked == v2, iota, LOGIT_PAD), axis=-1,
                 keepdims=True)
    w1 = jax.nn.sigmoid(v1 - v2)
    w2 = 1.0 - w1
    w = jnp.where(iota == i1, w1, 0.0) + jnp.where(iota == i2, w2, 0.0)
    h_ref[...] = h.astype(jnp.bfloat16)
    w_ref[...] = w


def _expert_kernel(h_ref, ef_ref, wcol_ref, we1a_ref, we1b_ref, be1_ref,
                   we2_ref, be2_ref, we3_ref, be3_ref, alpha_ref,
                   out_ref):
    e = pl.program_id(1)
    x1 = jnp.dot(h_ref[...], we1a_ref[0],
                 preferred_element_type=jnp.float32)
    x1 = x1 + jnp.dot(ef_ref[0], we1b_ref[0],
                      preferred_element_type=jnp.float32)
    h1 = _gelu_exact(x1 + be1_ref[0])
    h2 = jnp.dot(h1.astype(jnp.bfloat16), we2_ref[0],
                 preferred_element_type=jnp.float32)
    h2 = _gelu_exact(h2 + be2_ref[0])
    oe = jnp.dot(h2.astype(jnp.bfloat16), we3_ref[0],
                 preferred_element_type=jnp.float32)
    oe = oe + be3_ref[0]
    contrib = oe * (alpha_ref[0, 0] * wcol_ref[0])

    @pl.when(e == 0)
    def _init():
        out_ref[...] = contrib

    @pl.when(e > 0)
    def _acc():
        out_ref[...] = out_ref[...] + contrib


def kernel(hidden, feature_bank, expert_bank_idx, ln_gamma, ln_beta,
           rW1, rb1, rW2, rb2, We1, be1, We2, be2, We3, be3, alpha):
    feats = feature_bank.reshape(B, NC * FB)
    rW2p = jnp.zeros((RH, LOGIT_PAD), jnp.float32).at[:, :E].set(rW2)
    rb2p = jnp.full((1, LOGIT_PAD), -1e30, jnp.float32).at[0, :E].set(rb2)
    h_bf, w = pl.pallas_call(
        _router_kernel,
        grid=(B // TBA,),
        in_specs=[
            pl.BlockSpec((TBA, D), lambda i: (i, 0)),
            pl.BlockSpec((TBA, NC * FB), lambda i: (i, 0)),
            pl.BlockSpec((1, D), lambda i: (0, 0)),
            pl.BlockSpec((1, D), lambda i: (0, 0)),
            pl.BlockSpec((D, RH), lambda i: (0, 0)),
            pl.BlockSpec((NC * FB, RH), lambda i: (0, 0)),
            pl.BlockSpec((1, RH), lambda i: (0, 0)),
            pl.BlockSpec((RH, LOGIT_PAD), lambda i: (0, 0)),
            pl.BlockSpec((1, LOGIT_PAD), lambda i: (0, 0)),
        ],
        out_specs=[
            pl.BlockSpec((TBA, D), lambda i: (i, 0)),
            pl.BlockSpec((TBA, LOGIT_PAD), lambda i: (i, 0)),
        ],
        out_shape=[
            jax.ShapeDtypeStruct((B, D), jnp.bfloat16),
            jax.ShapeDtypeStruct((B, LOGIT_PAD), jnp.float32),
        ],
    )(hidden, feats, ln_gamma.reshape(1, D), ln_beta.reshape(1, D),
      rW1[:D], rW1[D:], rb1.reshape(1, RH), rW2p, rb2p)

    ef = jnp.take(feature_bank, expert_bank_idx.reshape(-1), axis=1)
    ef = ef.reshape(B, E, FPE * FB).transpose(1, 0, 2).astype(jnp.bfloat16)
    wcol = w[:, :E].T.reshape(E, B, 1)

    We1b16 = We1.astype(jnp.bfloat16)
    We2b16 = We2.astype(jnp.bfloat16)
    We3b16 = We3.astype(jnp.bfloat16)
    combined = pl.pallas_call(
        _expert_kernel,
        grid=(B // TBB, E),
        in_specs=[
            pl.BlockSpec((TBB, D), lambda t, e: (t, 0)),
            pl.BlockSpec((1, TBB, FPE * FB), lambda t, e: (e, t, 0)),
            pl.BlockSpec((1, TBB, 1), lambda t, e: (e, t, 0)),
            pl.BlockSpec((1, D, H), lambda t, e: (e, 0, 0)),
            pl.BlockSpec((1, FPE * FB, H), lambda t, e: (e, D // (FPE * FB), 0)),
            pl.BlockSpec((1, 1, H), lambda t, e: (e, 0, 0)),
            pl.BlockSpec((1, H, H), lambda t, e: (e, 0, 0)),
            pl.BlockSpec((1, 1, H), lambda t, e: (e, 0, 0)),
            pl.BlockSpec((1, H, D), lambda t, e: (e, 0, 0)),
            pl.BlockSpec((1, 1, D), lambda t, e: (e, 0, 0)),
            pl.BlockSpec((1, 1), lambda t, e: (0, 0)),
        ],
        out_specs=pl.BlockSpec((TBB, D), lambda t, e: (t, 0)),
        out_shape=jax.ShapeDtypeStruct((B, D), jnp.float32),
    )(h_bf, ef, wcol, We1b16, We1b16, be1.reshape(E, 1, H), We2b16,
      be2.reshape(E, 1, H), We3b16, be3.reshape(E, 1, D),
      alpha.reshape(1, 1))
    return hidden + combined
